# baseline (speedup 1.0000x reference)
import functools

import jax
import jax.numpy as jnp
from jax import lax
from jax.experimental import pallas as pl
from jax.experimental.pallas import tpu as pltpu, tpu_sc as plsc

LANES = 16
RES_PER_CHUNK = 2
NBUF = 8


@functools.lru_cache(maxsize=None)
def _make_kernel(n_res: int, t: int, d: int):
    info = plsc.get_sparse_core_info()
    nw = info.num_cores * info.num_subcores
    res_per_w = n_res // nw
    chunks_per_w = res_per_w // RES_PER_CHUNK
    idx_per_chunk = RES_PER_CHUNK * t

    mesh = plsc.VectorSubcoreMesh(core_axis_name="c", subcore_axis_name="s")

    @functools.partial(
        pl.kernel,
        mesh=mesh,
        out_type=jax.ShapeDtypeStruct((n_res, d), jnp.float32),
        scratch_types=[
            pltpu.VMEM((chunks_per_w, idx_per_chunk), jnp.int32),
            pltpu.VMEM((NBUF, idx_per_chunk, d), jnp.float32),
            pltpu.VMEM((res_per_w, d), jnp.float32),
        ] + [pltpu.SemaphoreType.DMA] * NBUF,
    )
    def k(idx_hbm, table_hbm, out_hbm, idx_v, rows_v, out_v, *sems):
        wid = lax.axis_index("s") * info.num_cores + lax.axis_index("c")
        pltpu.sync_copy(idx_hbm.at[pl.ds(wid * chunks_per_w, chunks_per_w)],
                        idx_v)

        def dma(j, b):
            return pltpu.make_async_copy(
                table_hbm.at[idx_v.at[j]], rows_v.at[b], sems[b])

        for b in range(NBUF - 1):
            dma(b, b).start()

        def body(g, _):
            for b in range(NBUF):
                j = g * NBUF + b
                nxt = j + NBUF - 1

                @pl.when(nxt < chunks_per_w)
                def _():
                    dma(nxt, (b + NBUF - 1) % NBUF).start()

                dma(j, b).wait()
                for r in range(RES_PER_CHUNK):
                    for dc in range(d // LANES):
                        sl = pl.ds(dc * LANES, LANES)
                        vals = [rows_v[b, r * t + tt, sl] for tt in range(t)]
                        while len(vals) > 1:
                            nxt_vals = [vals[i] + vals[i + 1]
                                        for i in range(0, len(vals) - 1, 2)]
                            if len(vals) % 2:
                                nxt_vals.append(vals[-1])
                            vals = nxt_vals
                        out_v[j * RES_PER_CHUNK + r, sl] = vals[0]
            return 0

        lax.fori_loop(0, chunks_per_w // NBUF, body, 0)
        pltpu.sync_copy(out_v, out_hbm.at[pl.ds(wid * res_per_w, res_per_w)])

    return k


def kernel(go_terms, table):
    b, l, t = go_terms.shape
    d = table.shape[1]
    n_res = b * l
    idx = go_terms.reshape(n_res // RES_PER_CHUNK, RES_PER_CHUNK * t)
    out = _make_kernel(n_res, t, d)(idx, table)
    return out.reshape(b, l, d)


# speedup vs baseline: 1.5367x; 1.5367x over previous
import functools

import jax
import jax.numpy as jnp
from jax import lax
from jax.experimental import pallas as pl
from jax.experimental.pallas import tpu as pltpu, tpu_sc as plsc

LANES = 16
RES_PER_CHUNK = 1
NBUF = 4


@functools.lru_cache(maxsize=None)
def _make_kernel(n_res: int, t: int, d: int):
    info = plsc.get_sparse_core_info()
    nw = info.num_cores * info.num_subcores
    res_per_w = n_res // nw
    chunks_per_w = res_per_w // RES_PER_CHUNK
    idx_per_chunk = RES_PER_CHUNK * t

    mesh = plsc.VectorSubcoreMesh(core_axis_name="c", subcore_axis_name="s")

    @functools.partial(
        pl.kernel,
        mesh=mesh,
        out_type=jax.ShapeDtypeStruct((n_res, d), jnp.float32),
        scratch_types=[
            pltpu.VMEM((chunks_per_w, idx_per_chunk), jnp.int32),
            pltpu.VMEM((NBUF, idx_per_chunk, d), jnp.float32),
            pltpu.VMEM((res_per_w, d), jnp.float32),
        ] + [pltpu.SemaphoreType.DMA] * NBUF,
    )
    def k(idx_hbm, table_hbm, out_hbm, idx_v, rows_v, out_v, *sems):
        wid = lax.axis_index("s") * info.num_cores + lax.axis_index("c")
        pltpu.sync_copy(idx_hbm.at[pl.ds(wid * chunks_per_w, chunks_per_w)],
                        idx_v)

        def dma(j, b):
            return pltpu.make_async_copy(
                table_hbm.at[idx_v.at[j]], rows_v.at[b], sems[b])

        for b in range(NBUF - 1):
            dma(b, b).start()

        def body(g, _):
            for b in range(NBUF):
                j = g * NBUF + b
                nxt = j + NBUF - 1

                @pl.when(nxt < chunks_per_w)
                def _():
                    dma(nxt, (b + NBUF - 1) % NBUF).start()

                dma(j, b).wait()
                for r in range(RES_PER_CHUNK):
                    for dc in range(d // LANES):
                        sl = pl.ds(dc * LANES, LANES)
                        vals = [rows_v[b, r * t + tt, sl] for tt in range(t)]
                        while len(vals) > 1:
                            nxt_vals = [vals[i] + vals[i + 1]
                                        for i in range(0, len(vals) - 1, 2)]
                            if len(vals) % 2:
                                nxt_vals.append(vals[-1])
                            vals = nxt_vals
                        out_v[j * RES_PER_CHUNK + r, sl] = vals[0]
            return 0

        lax.fori_loop(0, chunks_per_w // NBUF, body, 0)
        pltpu.sync_copy(out_v, out_hbm.at[pl.ds(wid * res_per_w, res_per_w)])

    return k


def kernel(go_terms, table):
    b, l, t = go_terms.shape
    d = table.shape[1]
    n_res = b * l
    idx = go_terms.reshape(n_res // RES_PER_CHUNK, RES_PER_CHUNK * t)
    out = _make_kernel(n_res, t, d)(idx, table)
    return out.reshape(b, l, d)
